# Initial kernel scaffold; baseline (speedup 1.0000x reference)
import jax
import jax.numpy as jnp
from jax.experimental import pallas as pl
from jax.experimental.pallas import tpu as pltpu
from jax.experimental.pallas import tpu_sc as plsc

VOCAB = 1000000
D_MODEL = 64
BATCH = 16384
HIST = 50
N_IDX = BATCH * HIST

IDX_MINOR = 128
ROWS_PER_STEP = 512
K = ROWS_PER_STEP // IDX_MINOR
GRID = N_IDX // ROWS_PER_STEP

_mesh = plsc.VectorSubcoreMesh(core_axis_name="core", subcore_axis_name="subcore")


@jax.jit
def _gather(table, idx3):
    @pl.kernel(
        out_type=jax.ShapeDtypeStruct((N_IDX, D_MODEL), jnp.float32),
        mesh=_mesh,
    )
    def kern(table_hbm, i_hbm, o_hbm):
        def body(i_vmem, o_vmem):
            for j in range(K):
                pltpu.sync_copy(
                    table_hbm.at[i_vmem.at[0, j]],
                    o_vmem.at[pl.ds(j * IDX_MINOR, IDX_MINOR)],
                )

        pltpu.emit_pipeline(
            body,
            grid=(GRID,),
            in_specs=[
                pl.BlockSpec((1, K, IDX_MINOR), index_map=lambda i: (i, 0, 0))
            ],
            out_specs=[
                pl.BlockSpec((ROWS_PER_STEP, D_MODEL), index_map=lambda i: (i, 0))
            ],
            core_axis_name=("core", "subcore"),
            dimension_semantics=(pltpu.PARALLEL,),
        )(i_hbm, o_hbm)

    return kern(table, idx3)


def kernel(x, table):
    idx3 = x.reshape(GRID, K, IDX_MINOR)
    out = _gather(table, idx3)
    return out.reshape(BATCH, HIST, D_MODEL)


# speedup vs baseline: 1.7540x; 1.7540x over previous
import jax
import jax.numpy as jnp
from jax.experimental import pallas as pl
from jax.experimental.pallas import tpu as pltpu
from jax.experimental.pallas import tpu_sc as plsc

VOCAB = 1000000
D_MODEL = 64
BATCH = 16384
HIST = 50
N_IDX = BATCH * HIST

IDX_MINOR = 128
ROWS_PER_STEP = 512
K = ROWS_PER_STEP // IDX_MINOR
GRID = N_IDX // ROWS_PER_STEP

_mesh = plsc.VectorSubcoreMesh(core_axis_name="core", subcore_axis_name="subcore")


@jax.jit
def _gather(table, idx3):
    @pl.kernel(
        out_type=jax.ShapeDtypeStruct((N_IDX, D_MODEL), jnp.float32),
        mesh=_mesh,
        compiler_params=pltpu.CompilerParams(use_tc_tiling_on_sc=False),
    )
    def kern(table_hbm, i_hbm, o_hbm):
        def body(i_vmem, o_vmem):
            for j in range(K):
                pltpu.sync_copy(
                    table_hbm.at[i_vmem.at[0, j]],
                    o_vmem.at[pl.ds(j * IDX_MINOR, IDX_MINOR)],
                )

        pltpu.emit_pipeline(
            body,
            grid=(GRID,),
            in_specs=[
                pl.BlockSpec((1, K, IDX_MINOR), index_map=lambda i: (i, 0, 0))
            ],
            out_specs=[
                pl.BlockSpec((ROWS_PER_STEP, D_MODEL), index_map=lambda i: (i, 0))
            ],
            core_axis_name=("core", "subcore"),
            dimension_semantics=(pltpu.PARALLEL,),
        )(i_hbm, o_hbm)

    return kern(table, idx3)


def kernel(x, table):
    idx3 = x.reshape(GRID, K, IDX_MINOR)
    out = _gather(table, idx3)
    return out.reshape(BATCH, HIST, D_MODEL)


# speedup vs baseline: 1.8677x; 1.0648x over previous
import jax
import jax.numpy as jnp
from jax.experimental import pallas as pl
from jax.experimental.pallas import tpu as pltpu
from jax.experimental.pallas import tpu_sc as plsc

VOCAB = 1000000
D_MODEL = 64
BATCH = 16384
HIST = 50
N_IDX = BATCH * HIST

IDX_MINOR = 128
ROWS_PER_STEP = 512
K = ROWS_PER_STEP // IDX_MINOR
GRID = N_IDX // ROWS_PER_STEP

_mesh = plsc.VectorSubcoreMesh(core_axis_name="core", subcore_axis_name="subcore")


@jax.jit
def _gather(table, idx3):
    @pl.kernel(
        out_type=jax.ShapeDtypeStruct((N_IDX, D_MODEL), jnp.float32),
        mesh=_mesh,
        compiler_params=pltpu.CompilerParams(use_tc_tiling_on_sc=False),
    )
    def kern(table_hbm, i_hbm, o_hbm):
        def body(i_vmem, o_vmem):
            pltpu.sync_copy(table_hbm.at[i_vmem.at[0]], o_vmem)

        pltpu.emit_pipeline(
            body,
            grid=(GRID,),
            in_specs=[
                pl.BlockSpec((1, ROWS_PER_STEP), index_map=lambda i: (0, i))
            ],
            out_specs=[
                pl.BlockSpec((ROWS_PER_STEP, D_MODEL), index_map=lambda i: (i, 0))
            ],
            core_axis_name=("core", "subcore"),
            dimension_semantics=(pltpu.PARALLEL,),
        )(i_hbm, o_hbm)

    return kern(table, idx3)


def kernel(x, table):
    idx3 = x.reshape(1, N_IDX)
    out = _gather(table, idx3)
    return out.reshape(BATCH, HIST, D_MODEL)
